# baseline (speedup 1.0000x reference)
import jax
import jax.numpy as jnp
from jax import lax
from jax.experimental import pallas as pl
from jax.experimental.pallas import tpu as pltpu
from jax.experimental.pallas import tpu_sc as plsc

EMB = 32
BATCH = 16384

_NC = 2
_NS = 16
_NW = _NC * _NS
_BPW = BATCH // _NW
_L = 16
_CHUNK = 128
_NCHUNK = _BPW // _CHUNK
_BG = 16


def _mf_body(u_id_hbm, i_id_hbm, u_hi_hbm, i_hi_hbm, u_lo_hbm, i_lo_hbm,
             ue_hbm, ub_hbm, ie_hbm, ib_hbm, mean_hbm, out_hbm,
             uidx_v, iidx_v, uhi_v, ihi_v, ulo_v, ilo_v,
             urows_v, irows_v, ubias_v, ibias_v, mean_v, out_v, sem):
    wid = lax.axis_index("s") * _NC + lax.axis_index("c")
    base = wid * _BPW

    pltpu.sync_copy(u_id_hbm.at[pl.ds(wid * _NCHUNK, _NCHUNK)], uidx_v)
    pltpu.sync_copy(i_id_hbm.at[pl.ds(wid * _NCHUNK, _NCHUNK)], iidx_v)
    pltpu.sync_copy(u_hi_hbm.at[pl.ds(wid * _NCHUNK, _NCHUNK)], uhi_v)
    pltpu.sync_copy(i_hi_hbm.at[pl.ds(wid * _NCHUNK, _NCHUNK)], ihi_v)
    pltpu.sync_copy(u_lo_hbm.at[pl.ds(base, _BPW)], ulo_v)
    pltpu.sync_copy(i_lo_hbm.at[pl.ds(base, _BPW)], ilo_v)
    copies = []
    for j in range(_NCHUNK):
        rs = pl.ds(j * _CHUNK, _CHUNK)
        copies.append(pltpu.async_copy(
            ue_hbm.at[uidx_v.at[j]], urows_v.at[rs], sem))
        copies.append(pltpu.async_copy(
            ie_hbm.at[iidx_v.at[j]], irows_v.at[rs], sem))
        copies.append(pltpu.async_copy(
            ub_hbm.at[uhi_v.at[j]], ubias_v.at[rs], sem))
        copies.append(pltpu.async_copy(
            ib_hbm.at[ihi_v.at[j]], ibias_v.at[rs], sem))
    pltpu.sync_copy(mean_hbm, mean_v)
    for c in copies:
        c.wait()

    mean_b = mean_v[...]

    def chunk(c, _):
        rows = c * _L + lax.iota(jnp.int32, _L)
        acc = jnp.zeros((_L,), jnp.float32)
        for e in range(EMB):
            col = jnp.full((_L,), e, jnp.int32)
            uv = plsc.load_gather(urows_v, [rows, col])
            iv = plsc.load_gather(irows_v, [rows, col])
            acc = acc + uv * iv
        bu = plsc.load_gather(ubias_v, [rows, ulo_v[pl.ds(c * _L, _L)]])
        bi = plsc.load_gather(ibias_v, [rows, ilo_v[pl.ds(c * _L, _L)]])
        out_v[pl.ds(c * _L, _L)] = acc + bu + bi + mean_b
        return _

    lax.fori_loop(0, _BPW // _L, chunk, None)
    pltpu.sync_copy(out_v, out_hbm.at[pl.ds(base, _BPW)])


@jax.jit
def _mf(u_id, i_id, user_emb, user_bias, item_emb, item_bias, mean):
    mesh = plsc.VectorSubcoreMesh(core_axis_name="c", subcore_axis_name="s")
    return pl.kernel(
        _mf_body,
        out_type=jax.ShapeDtypeStruct((BATCH,), jnp.float32),
        mesh=mesh,
        compiler_params=pltpu.CompilerParams(
            needs_layout_passes=False, use_tc_tiling_on_sc=False),
        scratch_types=[
            pltpu.VMEM((_NCHUNK, _CHUNK), jnp.int32),
            pltpu.VMEM((_NCHUNK, _CHUNK), jnp.int32),
            pltpu.VMEM((_NCHUNK, _CHUNK), jnp.int32),
            pltpu.VMEM((_NCHUNK, _CHUNK), jnp.int32),
            pltpu.VMEM((_BPW,), jnp.int32),
            pltpu.VMEM((_BPW,), jnp.int32),
            pltpu.VMEM((_BPW, EMB), jnp.float32),
            pltpu.VMEM((_BPW, EMB), jnp.float32),
            pltpu.VMEM((_BPW, _BG), jnp.float32),
            pltpu.VMEM((_BPW, _BG), jnp.float32),
            pltpu.VMEM((_L,), jnp.float32),
            pltpu.VMEM((_BPW,), jnp.float32),
            pltpu.SemaphoreType.DMA,
        ],
    )(u_id.reshape(BATCH // _CHUNK, _CHUNK),
      i_id.reshape(BATCH // _CHUNK, _CHUNK),
      (u_id >> 4).reshape(BATCH // _CHUNK, _CHUNK),
      (i_id >> 4).reshape(BATCH // _CHUNK, _CHUNK),
      u_id & (_BG - 1),
      i_id & (_BG - 1),
      user_emb,
      user_bias.reshape(user_bias.shape[0] // _BG, _BG),
      item_emb,
      item_bias.reshape(item_bias.shape[0] // _BG, _BG),
      jnp.broadcast_to(mean, (_L,)))


def kernel(u_id, i_id, user_emb, user_bias, item_emb, item_bias, mean):
    return _mf(u_id, i_id, user_emb, user_bias, item_emb, item_bias, mean)


# speedup vs baseline: 1.8067x; 1.8067x over previous
import jax
import jax.numpy as jnp
from jax import lax
from jax.experimental import pallas as pl
from jax.experimental.pallas import tpu as pltpu
from jax.experimental.pallas import tpu_sc as plsc

EMB = 32
BATCH = 16384

_NC = 2
_NS = 16
_NW = _NC * _NS
_BPW = BATCH // _NW
_L = 16
_NCH = _BPW // _L
_A = EMB // 8


def _mf_body(uet_hbm, iet_hbm, ub_hbm, ib_hbm, u_id_hbm, i_id_hbm, mean_hbm,
             out_hbm, uids_v, iids_v, ublk, iblk, bublk, biblk, mean_v,
             out_v, sem):
    wid = lax.axis_index("s") * _NC + lax.axis_index("c")
    base = wid * _BPW

    pltpu.sync_copy(u_id_hbm.at[pl.ds(base, _BPW)], uids_v)
    pltpu.sync_copy(i_id_hbm.at[pl.ds(base, _BPW)], iids_v)
    pltpu.sync_copy(mean_hbm, mean_v)
    mean_b = mean_v[...]
    iota = lax.iota(jnp.int32, _L)

    def chunk(c, _):
        idu = uids_v[pl.ds(c * _L, _L)]
        idi = iids_v[pl.ds(c * _L, _L)]
        copies = []
        for k in range(_L):
            gu = (idu[k] // _L) * _L
            gi = (idi[k] // _L) * _L
            for a in range(_A):
                for s in range(8):
                    copies.append(pltpu.async_copy(
                        uet_hbm.at[a, s, pl.ds(gu, _L)],
                        ublk.at[k * _A + a, pl.ds(s * _L, _L)], sem))
                    copies.append(pltpu.async_copy(
                        iet_hbm.at[a, s, pl.ds(gi, _L)],
                        iblk.at[k * _A + a, pl.ds(s * _L, _L)], sem))
            copies.append(pltpu.async_copy(
                ub_hbm.at[pl.ds(gu, _L)], bublk.at[k, pl.ds(0, _L)], sem))
            copies.append(pltpu.async_copy(
                ib_hbm.at[pl.ds(gi, _L)], biblk.at[k, pl.ds(0, _L)], sem))
        for cc in copies:
            cc.wait()

        lo_u = idu & (_L - 1)
        lo_i = idi & (_L - 1)
        rows = iota * _A
        acc = jnp.zeros((_L,), jnp.float32)
        for e in range(EMB):
            a, s = e // 8, e % 8
            uv = plsc.load_gather(ublk, [rows + a, lo_u + s * _L])
            iv = plsc.load_gather(iblk, [rows + a, lo_i + s * _L])
            acc = acc + uv * iv
        bu = plsc.load_gather(bublk, [iota, lo_u])
        bi = plsc.load_gather(biblk, [iota, lo_i])
        out_v[pl.ds(c * _L, _L)] = acc + bu + bi + mean_b
        return _

    lax.fori_loop(0, _NCH, chunk, None)
    pltpu.sync_copy(out_v, out_hbm.at[pl.ds(base, _BPW)])


@jax.jit
def _mf(u_id, i_id, user_emb, user_bias, item_emb, item_bias, mean):
    n = user_emb.shape[0]
    mesh = plsc.VectorSubcoreMesh(core_axis_name="c", subcore_axis_name="s")
    return pl.kernel(
        _mf_body,
        out_type=jax.ShapeDtypeStruct((BATCH,), jnp.float32),
        mesh=mesh,
        compiler_params=pltpu.CompilerParams(
            needs_layout_passes=False, use_tc_tiling_on_sc=True),
        scratch_types=[
            pltpu.VMEM((_BPW,), jnp.int32),
            pltpu.VMEM((_BPW,), jnp.int32),
            pltpu.VMEM((_L * _A, 128), jnp.float32),
            pltpu.VMEM((_L * _A, 128), jnp.float32),
            pltpu.VMEM((_L, 128), jnp.float32),
            pltpu.VMEM((_L, 128), jnp.float32),
            pltpu.VMEM((_L,), jnp.float32),
            pltpu.VMEM((_BPW,), jnp.float32),
            pltpu.SemaphoreType.DMA,
        ],
    )(user_emb.T.reshape(_A, 8, n),
      item_emb.T.reshape(_A, 8, n),
      user_bias[:, 0],
      item_bias[:, 0],
      u_id, i_id,
      jnp.broadcast_to(mean, (_L,)))


def kernel(u_id, i_id, user_emb, user_bias, item_emb, item_bias, mean):
    return _mf(u_id, i_id, user_emb, user_bias, item_emb, item_bias, mean)


# speedup vs baseline: 5.2972x; 2.9321x over previous
import jax
import jax.numpy as jnp
from jax import lax
from jax.experimental import pallas as pl
from jax.experimental.pallas import tpu as pltpu
from jax.experimental.pallas import tpu_sc as plsc

EMB = 32
BATCH = 16384

_NC = 2
_NS = 16
_NW = _NC * _NS
_BPW = BATCH // _NW
_L = 16
_NCH = _BPW // _L
_A = EMB // 8


def _mf_body(uet_hbm, iet_hbm, ub_hbm, ib_hbm, u_id_hbm, i_id_hbm, mean_hbm,
             out_hbm, uids_v, iids_v,
             ublk0, iblk0, bublk0, biblk0,
             ublk1, iblk1, bublk1, biblk1,
             mean_v, out_v, sem0, sem1):
    wid = lax.axis_index("s") * _NC + lax.axis_index("c")
    base = wid * _BPW
    bufs = ((ublk0, iblk0, bublk0, biblk0, sem0),
            (ublk1, iblk1, bublk1, biblk1, sem1))

    pltpu.sync_copy(u_id_hbm.at[pl.ds(base, _BPW)], uids_v.at[pl.ds(0, _BPW)])
    pltpu.sync_copy(i_id_hbm.at[pl.ds(base, _BPW)], iids_v.at[pl.ds(0, _BPW)])
    pltpu.sync_copy(mean_hbm, mean_v)
    mean_b = mean_v[...]
    iota = lax.iota(jnp.int32, _L)

    def fire(c, par):
        ublk, iblk, bublk, biblk, sem = bufs[par]

        def fk(k, _):
            vu = uids_v[pl.ds(c * _L + k, _L)]
            vi = iids_v[pl.ds(c * _L + k, _L)]
            gu = (vu[0] // _L) * _L
            gi = (vi[0] // _L) * _L
            for a in range(_A):
                for s in range(8):
                    pltpu.async_copy(
                        uet_hbm.at[a, s, pl.ds(gu, _L)],
                        ublk.at[k * _A + a, pl.ds(s * _L, _L)], sem)
                    pltpu.async_copy(
                        iet_hbm.at[a, s, pl.ds(gi, _L)],
                        iblk.at[k * _A + a, pl.ds(s * _L, _L)], sem)
            pltpu.async_copy(
                ub_hbm.at[pl.ds(gu, _L)], bublk.at[k, pl.ds(0, _L)], sem)
            pltpu.async_copy(
                ib_hbm.at[pl.ds(gi, _L)], biblk.at[k, pl.ds(0, _L)], sem)
            return _

        lax.fori_loop(0, _L, fk, None)

    def drain(par):
        ublk, iblk, bublk, biblk, sem = bufs[par]
        for r in range(8):
            src = uet_hbm.at[0, pl.ds(0, 8), pl.ds(0, 128)]
            pltpu.make_async_copy(src, ublk.at[pl.ds(r * 8, 8)], sem).wait()
            pltpu.make_async_copy(src, iblk.at[pl.ds(r * 8, 8)], sem).wait()
        src2 = uet_hbm.at[0, pl.ds(0, 2), pl.ds(0, 128)]
        pltpu.make_async_copy(src2, bublk.at[pl.ds(0, 2)], sem).wait()
        pltpu.make_async_copy(src2, biblk.at[pl.ds(0, 2)], sem).wait()

    def extract(c, par):
        ublk, iblk, bublk, biblk, sem = bufs[par]
        idu = uids_v[pl.ds(c * _L, _L)]
        idi = iids_v[pl.ds(c * _L, _L)]
        lo_u = idu & (_L - 1)
        lo_i = idi & (_L - 1)
        rows = iota * _A
        acc = jnp.zeros((_L,), jnp.float32)
        for e in range(EMB):
            a, s = e // 8, e % 8
            uv = plsc.load_gather(ublk, [rows + a, lo_u + s * _L])
            iv = plsc.load_gather(iblk, [rows + a, lo_i + s * _L])
            acc = acc + uv * iv
        bu = plsc.load_gather(bublk, [iota, lo_u])
        bi = plsc.load_gather(biblk, [iota, lo_i])
        out_v[pl.ds(c * _L, _L)] = acc + bu + bi + mean_b

    fire(0, 0)

    def pipe(g, _):
        fire(2 * g + 1, 1)
        drain(0)
        extract(2 * g, 0)

        @pl.when(g < _NCH // 2 - 1)
        def _fire_next():
            fire(2 * g + 2, 0)

        drain(1)
        extract(2 * g + 1, 1)
        return _

    lax.fori_loop(0, _NCH // 2, pipe, None)
    pltpu.sync_copy(out_v, out_hbm.at[pl.ds(base, _BPW)])


@jax.jit
def _mf(u_id, i_id, user_emb, user_bias, item_emb, item_bias, mean):
    n = user_emb.shape[0]
    mesh = plsc.VectorSubcoreMesh(core_axis_name="c", subcore_axis_name="s")
    blk = lambda: pltpu.VMEM((_L * _A, 128), jnp.float32)
    bblk = lambda: pltpu.VMEM((_L, 128), jnp.float32)
    return pl.kernel(
        _mf_body,
        out_type=jax.ShapeDtypeStruct((BATCH,), jnp.float32),
        mesh=mesh,
        compiler_params=pltpu.CompilerParams(
            needs_layout_passes=False, use_tc_tiling_on_sc=True),
        scratch_types=[
            pltpu.VMEM((_BPW + _L,), jnp.int32),
            pltpu.VMEM((_BPW + _L,), jnp.int32),
            blk(), blk(), bblk(), bblk(),
            blk(), blk(), bblk(), bblk(),
            pltpu.VMEM((_L,), jnp.float32),
            pltpu.VMEM((_BPW,), jnp.float32),
            pltpu.SemaphoreType.DMA,
            pltpu.SemaphoreType.DMA,
        ],
    )(user_emb.T.reshape(_A, 8, n),
      item_emb.T.reshape(_A, 8, n),
      user_bias[:, 0],
      item_bias[:, 0],
      u_id, i_id,
      jnp.broadcast_to(mean, (_L,)))


def kernel(u_id, i_id, user_emb, user_bias, item_emb, item_bias, mean):
    return _mf(u_id, i_id, user_emb, user_bias, item_emb, item_bias, mean)


# speedup vs baseline: 10.5074x; 1.9836x over previous
import jax
import jax.numpy as jnp
from jax import lax
from jax.experimental import pallas as pl
from jax.experimental.pallas import tpu as pltpu
from jax.experimental.pallas import tpu_sc as plsc

EMB = 32
BATCH = 16384

_NC = 2
_NS = 16
_NW = _NC * _NS
_BPW = BATCH // _NW
_L = 16
_NCH = _BPW // _L
_A = EMB // 8


def _mf_body(uet_hbm, iet_hbm, ub_hbm, ib_hbm, u_id_hbm, i_id_hbm, mean_hbm,
             out_hbm, uids_v, iids_v,
             ublk0, iblk0, bublk0, biblk0,
             ublk1, iblk1, bublk1, biblk1,
             mean_v, out_v, sem0, sem1):
    wid = lax.axis_index("s") * _NC + lax.axis_index("c")
    base = wid * _BPW
    bufs = ((ublk0, iblk0, bublk0, biblk0, sem0),
            (ublk1, iblk1, bublk1, biblk1, sem1))

    pltpu.sync_copy(u_id_hbm.at[pl.ds(base, _BPW)], uids_v.at[pl.ds(0, _BPW)])
    pltpu.sync_copy(i_id_hbm.at[pl.ds(base, _BPW)], iids_v.at[pl.ds(0, _BPW)])
    pltpu.sync_copy(mean_hbm, mean_v)
    mean_b = mean_v[...]
    iota = lax.iota(jnp.int32, _L)

    def fire(c, par):
        ublk, iblk, bublk, biblk, sem = bufs[par]

        def fk(k, _):
            vu = uids_v[pl.ds(c * _L + k, _L)]
            vi = iids_v[pl.ds(c * _L + k, _L)]
            gu = (vu[0] // _L) * _L
            gi = (vi[0] // _L) * _L
            for a in range(_A):
                for s in range(8):
                    pltpu.async_copy(
                        uet_hbm.at[a, s, pl.ds(gu, _L)],
                        ublk.at[k * _A + a, pl.ds(s * _L, _L)], sem)
                    pltpu.async_copy(
                        iet_hbm.at[a, s, pl.ds(gi, _L)],
                        iblk.at[k * _A + a, pl.ds(s * _L, _L)], sem)
            pltpu.async_copy(
                ub_hbm.at[0, pl.ds(gu, _L)], bublk.at[k, pl.ds(0, _L)], sem)
            pltpu.async_copy(
                ib_hbm.at[0, pl.ds(gi, _L)], biblk.at[k, pl.ds(0, _L)], sem)
            return _

        lax.fori_loop(0, _L, fk, None)

    def drain(par):
        ublk, iblk, bublk, biblk, sem = bufs[par]
        for r in range(8):
            src = uet_hbm.at[0, pl.ds(0, 8), pl.ds(0, 128)]
            pltpu.make_async_copy(src, ublk.at[pl.ds(r * 8, 8)], sem).wait()
            pltpu.make_async_copy(src, iblk.at[pl.ds(r * 8, 8)], sem).wait()
        src2 = uet_hbm.at[0, pl.ds(0, 2), pl.ds(0, 128)]
        pltpu.make_async_copy(src2, bublk.at[pl.ds(0, 2)], sem).wait()
        pltpu.make_async_copy(src2, biblk.at[pl.ds(0, 2)], sem).wait()

    def extract(c, par):
        ublk, iblk, bublk, biblk, sem = bufs[par]
        idu = uids_v[pl.ds(c * _L, _L)]
        idi = iids_v[pl.ds(c * _L, _L)]
        lo_u = idu & (_L - 1)
        lo_i = idi & (_L - 1)
        rows = iota * _A
        acc = jnp.zeros((_L,), jnp.float32)
        for e in range(EMB):
            a, s = e // 8, e % 8
            uv = plsc.load_gather(ublk, [rows + a, lo_u + s * _L])
            iv = plsc.load_gather(iblk, [rows + a, lo_i + s * _L])
            acc = acc + uv * iv
        bu = plsc.load_gather(bublk, [iota, lo_u])
        bi = plsc.load_gather(biblk, [iota, lo_i])
        out_v[pl.ds(c * _L, _L)] = acc + bu + bi + mean_b

    fire(0, 0)

    def pipe(g, _):
        fire(2 * g + 1, 1)
        drain(0)
        extract(2 * g, 0)

        @pl.when(g < _NCH // 2 - 1)
        def _fire_next():
            fire(2 * g + 2, 0)

        drain(1)
        extract(2 * g + 1, 1)
        return _

    lax.fori_loop(0, _NCH // 2, pipe, None)
    pltpu.sync_copy(out_v, out_hbm.at[pl.ds(base, _BPW)])


@jax.jit
def _mf(u_id, i_id, user_emb, user_bias, item_emb, item_bias, mean):
    n = user_emb.shape[0]
    mesh = plsc.VectorSubcoreMesh(core_axis_name="c", subcore_axis_name="s")
    blk = lambda: pltpu.VMEM((_L * _A, 128), jnp.float32)
    bblk = lambda: pltpu.VMEM((_L, 128), jnp.float32)
    return pl.kernel(
        _mf_body,
        out_type=jax.ShapeDtypeStruct((BATCH,), jnp.float32),
        mesh=mesh,
        compiler_params=pltpu.CompilerParams(
            needs_layout_passes=False, use_tc_tiling_on_sc=True),
        scratch_types=[
            pltpu.VMEM((_BPW + _L,), jnp.int32),
            pltpu.VMEM((_BPW + _L,), jnp.int32),
            blk(), blk(), bblk(), bblk(),
            blk(), blk(), bblk(), bblk(),
            pltpu.VMEM((_L,), jnp.float32),
            pltpu.VMEM((_BPW,), jnp.float32),
            pltpu.SemaphoreType.DMA,
            pltpu.SemaphoreType.DMA,
        ],
    )(user_emb.T.reshape(_A, 8, n),
      item_emb.T.reshape(_A, 8, n),
      user_bias.T,
      item_bias.T,
      u_id, i_id,
      jnp.broadcast_to(mean, (_L,)))


def kernel(u_id, i_id, user_emb, user_bias, item_emb, item_bias, mean):
    return _mf(u_id, i_id, user_emb, user_bias, item_emb, item_bias, mean)
